# baseline (speedup 1.0000x reference)
import functools

import jax
import jax.numpy as jnp
from jax import lax
from jax.experimental import pallas as pl
from jax.experimental.pallas import tpu as pltpu
from jax.experimental.pallas import tpu_sc as plsc

VOCAB = 1000000
D = 64
NUM_CLASS = 20
B = 16384
HIST = 50

K = 128
KACC = 32

NC = 2
NS = 16
LANES = 16
NW = NC * NS
BPW = B // NW
BAGS_PER_CHUNK = 2
CIDX = BAGS_PER_CHUNK * HIST
NCH = BPW // BAGS_PER_CHUNK


def _proj_body(et_ref, w_ref, b_ref, o_ref):
    p = (
        lax.dot_general(et_ref[...], w_ref[...], (((0,), (0,)), ((), ())),
                        preferred_element_type=jnp.float32)
        + b_ref[...]
    )
    o_ref[...] = jnp.concatenate(
        [p[q * (_VB // 4):(q + 1) * (_VB // 4), :] for q in range(4)], axis=1)


_VB = 32768
_NB = pl.cdiv(VOCAB, _VB)
VOCAB_PAD = _NB * _VB
_proj = pl.pallas_call(
    _proj_body,
    grid=(_NB,),
    in_specs=[
        pl.BlockSpec((D, _VB), lambda i: (0, i)),
        pl.BlockSpec((D, KACC), lambda i: (0, 0)),
        pl.BlockSpec((1, KACC), lambda i: (0, 0)),
    ],
    out_specs=pl.BlockSpec((_VB // 4, 4 * KACC), lambda i: (i, 0)),
    out_shape=jax.ShapeDtypeStruct((VOCAB_PAD // 4, 4 * KACC), jnp.float32),
)


def _sc_body(text_hbm, table_hbm, dummy_hbm, out_hbm, idx_v, rows0, rows1,
             rows2, rows3, rows4, rows5, rows6, rows7, out_v,
             sem0, sem1, sem2, sem3, sem4, sem5, sem6, sem7):
    wid = lax.axis_index("s") * NC + lax.axis_index("c")

    pltpu.sync_copy(text_hbm.at[wid], idx_v)

    def start(chunk, rows, sem):
        pltpu.async_copy(table_hbm.at[idx_v.at[chunk]], rows, sem)

    def drain(rows, sem):
        pltpu.make_async_copy(dummy_hbm, rows, sem).wait()

    def accumulate(rows, chunk):
        for bag in range(BAGS_PER_CHUNK):
            def row_body(j, accs):
                r = bag * HIST + j
                return tuple(
                    accs[k] + rows[r, pl.ds(k * LANES, LANES)]
                    for k in range(KACC // LANES)
                )
            accs = tuple(jnp.zeros((LANES,), jnp.float32)
                         for _ in range(KACC // LANES))
            accs = lax.fori_loop(0, HIST, row_body, accs)
            b_local = chunk * BAGS_PER_CHUNK + bag
            for k in range(KACC // LANES):
                out_v[b_local, pl.ds(k * LANES, LANES)] = accs[k]

    bufs = ((rows0, sem0), (rows1, sem1), (rows2, sem2), (rows3, sem3),
            (rows4, sem4), (rows5, sem5), (rows6, sem6), (rows7, sem7))
    nbuf = len(bufs)

    for k, (rows, sem) in enumerate(bufs):
        start(k, rows, sem)

    def body(c, _):
        for k, (rows, sem) in enumerate(bufs):
            drain(rows, sem)
            accumulate(rows, c + k)

            @pl.when(c + k + nbuf < NCH)
            def _():
                start(c + k + nbuf, rows, sem)
        return 0

    lax.fori_loop(0, NCH // nbuf, lambda i, carry: body(i * nbuf, carry), 0)

    pltpu.sync_copy(out_v, out_hbm.at[wid])


_sc_sum = functools.partial(
    pl.kernel,
    mesh=plsc.VectorSubcoreMesh(core_axis_name="c", subcore_axis_name="s",
                                num_cores=NC, num_subcores=NS),
    out_type=jax.ShapeDtypeStruct((NW, BPW, KACC), jnp.float32),
    scratch_types=[
        pltpu.VMEM((NCH, CIDX), jnp.int32),
        pltpu.VMEM((CIDX, KACC), jnp.float32),
        pltpu.VMEM((CIDX, KACC), jnp.float32),
        pltpu.VMEM((CIDX, KACC), jnp.float32),
        pltpu.VMEM((CIDX, KACC), jnp.float32),
        pltpu.VMEM((CIDX, KACC), jnp.float32),
        pltpu.VMEM((CIDX, KACC), jnp.float32),
        pltpu.VMEM((CIDX, KACC), jnp.float32),
        pltpu.VMEM((CIDX, KACC), jnp.float32),
        pltpu.VMEM((BPW, KACC), jnp.float32),
        pltpu.SemaphoreType.DMA,
        pltpu.SemaphoreType.DMA,
        pltpu.SemaphoreType.DMA,
        pltpu.SemaphoreType.DMA,
        pltpu.SemaphoreType.DMA,
        pltpu.SemaphoreType.DMA,
        pltpu.SemaphoreType.DMA,
        pltpu.SemaphoreType.DMA,
    ],
    compiler_params=pltpu.CompilerParams(use_tc_tiling_on_sc=False),
)(_sc_body)


def kernel(text, emb_weight, fc_weight, fc_bias):
    w32 = jnp.zeros((D, KACC), jnp.float32).at[:, :NUM_CLASS].set(
        fc_weight.T * (1.0 / HIST))
    b32 = jnp.zeros((1, KACC), jnp.float32).at[0, :NUM_CLASS].set(
        fc_bias * (1.0 / HIST))
    table = _proj(emb_weight.T, w32, b32).reshape(VOCAB_PAD, KACC)
    q_shift = (_VB // 4).bit_length() - 1
    v = text.astype(jnp.int32)
    w = (v & ~(_VB - 1)) + ((v & (_VB // 4 - 1)) << 2) + ((v >> q_shift) & 3)
    idx = w.reshape(NW, NCH, CIDX)
    dummy = jnp.zeros((CIDX, KACC), jnp.float32)
    sums = _sc_sum(idx, table, dummy).reshape(B, KACC)
    return sums[:, :NUM_CLASS]


# speedup vs baseline: 1.0788x; 1.0788x over previous
import functools

import jax
import jax.numpy as jnp
from jax import lax
from jax.experimental import pallas as pl
from jax.experimental.pallas import tpu as pltpu
from jax.experimental.pallas import tpu_sc as plsc

VOCAB = 1000000
D = 64
NUM_CLASS = 20
B = 16384
HIST = 50

K = 128
KACC = 32

NC = 2
NS = 16
LANES = 16
NW = NC * NS
BPW = B // NW
BAGS_PER_CHUNK = 2
CIDX = BAGS_PER_CHUNK * HIST
NCH = BPW // BAGS_PER_CHUNK


def _proj_body(et_ref, w_ref, b_ref, o_ref):
    p = (
        lax.dot_general(et_ref[...], w_ref[...], (((0,), (0,)), ((), ())),
                        preferred_element_type=jnp.float32)
        + b_ref[...]
    )
    o_ref[...] = jnp.concatenate(
        [p[q * (_VB // 4):(q + 1) * (_VB // 4), :] for q in range(4)], axis=1)


_VB = 32768
_NB = pl.cdiv(VOCAB, _VB)
VOCAB_PAD = _NB * _VB
_proj = pl.pallas_call(
    _proj_body,
    grid=(_NB,),
    in_specs=[
        pl.BlockSpec((D, _VB), lambda i: (0, i)),
        pl.BlockSpec((D, KACC), lambda i: (0, 0)),
        pl.BlockSpec((1, KACC), lambda i: (0, 0)),
    ],
    out_specs=pl.BlockSpec((_VB // 4, 4 * KACC), lambda i: (i, 0)),
    out_shape=jax.ShapeDtypeStruct((VOCAB_PAD // 4, 4 * KACC), jnp.float32),
)


def _sc_body(text_hbm, table_hbm, dummy_hbm, out_hbm, idx_v, rows0, rows1,
             rows2, rows3, out_v, sem0, sem1, sem2, sem3):
    wid = lax.axis_index("s") * NC + lax.axis_index("c")

    pltpu.sync_copy(text_hbm.at[wid], idx_v)

    def start(chunk, rows, sem):
        pltpu.async_copy(table_hbm.at[idx_v.at[chunk]], rows, sem)

    def drain(rows, sem):
        pltpu.make_async_copy(dummy_hbm, rows, sem).wait()

    def accumulate(rows, chunk):
        for bag in range(BAGS_PER_CHUNK):
            accs = [jnp.zeros((LANES,), jnp.float32)
                    for _ in range(KACC // LANES)]
            for j in range(HIST):
                for k in range(KACC // LANES):
                    accs[k] = accs[k] + rows[bag * HIST + j,
                                             pl.ds(k * LANES, LANES)]
            b_local = chunk * BAGS_PER_CHUNK + bag
            for k in range(KACC // LANES):
                out_v[b_local, pl.ds(k * LANES, LANES)] = accs[k]

    bufs = ((rows0, sem0), (rows1, sem1), (rows2, sem2), (rows3, sem3))
    nbuf = len(bufs)

    for k, (rows, sem) in enumerate(bufs):
        start(k, rows, sem)

    def body(c, _):
        for k, (rows, sem) in enumerate(bufs):
            drain(rows, sem)
            accumulate(rows, c + k)

            @pl.when(c + k + nbuf < NCH)
            def _():
                start(c + k + nbuf, rows, sem)
        return 0

    lax.fori_loop(0, NCH // nbuf, lambda i, carry: body(i * nbuf, carry), 0)

    pltpu.sync_copy(out_v, out_hbm.at[wid])


_sc_sum = functools.partial(
    pl.kernel,
    mesh=plsc.VectorSubcoreMesh(core_axis_name="c", subcore_axis_name="s",
                                num_cores=NC, num_subcores=NS),
    out_type=jax.ShapeDtypeStruct((NW, BPW, KACC), jnp.float32),
    scratch_types=[
        pltpu.VMEM((NCH, CIDX), jnp.int32),
        pltpu.VMEM((CIDX, KACC), jnp.float32),
        pltpu.VMEM((CIDX, KACC), jnp.float32),
        pltpu.VMEM((CIDX, KACC), jnp.float32),
        pltpu.VMEM((CIDX, KACC), jnp.float32),
        pltpu.VMEM((BPW, KACC), jnp.float32),
        pltpu.SemaphoreType.DMA,
        pltpu.SemaphoreType.DMA,
        pltpu.SemaphoreType.DMA,
        pltpu.SemaphoreType.DMA,
    ],
    compiler_params=pltpu.CompilerParams(use_tc_tiling_on_sc=False),
)(_sc_body)


def kernel(text, emb_weight, fc_weight, fc_bias):
    w32 = jnp.zeros((D, KACC), jnp.float32).at[:, :NUM_CLASS].set(
        fc_weight.T * (1.0 / HIST))
    b32 = jnp.zeros((1, KACC), jnp.float32).at[0, :NUM_CLASS].set(
        fc_bias * (1.0 / HIST))
    table = _proj(emb_weight.T, w32, b32).reshape(VOCAB_PAD, KACC)
    q_shift = (_VB // 4).bit_length() - 1
    v = text.astype(jnp.int32)
    w = (v & ~(_VB - 1)) + ((v & (_VB // 4 - 1)) << 2) + ((v >> q_shift) & 3)
    idx = w.reshape(NW, NCH, CIDX)
    dummy = jnp.zeros((CIDX, KACC), jnp.float32)
    sums = _sc_sum(idx, table, dummy).reshape(B, KACC)
    return sums[:, :NUM_CLASS]


# speedup vs baseline: 1.0797x; 1.0008x over previous
import functools

import jax
import jax.numpy as jnp
from jax import lax
from jax.experimental import pallas as pl
from jax.experimental.pallas import tpu as pltpu
from jax.experimental.pallas import tpu_sc as plsc

VOCAB = 1000000
D = 64
NUM_CLASS = 20
B = 16384
HIST = 50

K = 128
KACC = 32

NC = 2
NS = 16
LANES = 16
NW = NC * NS
BPW = B // NW
BAGS_PER_CHUNK = 2
CIDX = BAGS_PER_CHUNK * HIST
NCH = BPW // BAGS_PER_CHUNK


def _proj_body(et_ref, w_ref, b_ref, o_ref):
    p = (
        lax.dot_general(et_ref[...], w_ref[...], (((0,), (0,)), ((), ())),
                        preferred_element_type=jnp.float32)
        + b_ref[...]
    )
    o_ref[...] = jnp.concatenate(
        [p[q * (_VB // 4):(q + 1) * (_VB // 4), :] for q in range(4)], axis=1)


_VB = 32768
_NB = pl.cdiv(VOCAB, _VB)
VOCAB_PAD = _NB * _VB
_proj = pl.pallas_call(
    _proj_body,
    grid=(_NB,),
    in_specs=[
        pl.BlockSpec((D, _VB), lambda i: (0, i)),
        pl.BlockSpec((D, KACC), lambda i: (0, 0)),
        pl.BlockSpec((1, KACC), lambda i: (0, 0)),
    ],
    out_specs=pl.BlockSpec((_VB // 4, 4 * KACC), lambda i: (i, 0)),
    out_shape=jax.ShapeDtypeStruct((VOCAB_PAD // 4, 4 * KACC), jnp.float32),
)


def _sc_body(text_hbm, table_hbm, dummy_hbm, out_hbm, idx_v, rows0, rows1,
             rows2, rows3, out_v, sem0, sem1, sem2, sem3):
    wid = lax.axis_index("s") * NC + lax.axis_index("c")

    pltpu.sync_copy(text_hbm.at[wid], idx_v)

    def start(chunk, rows, sem):
        pltpu.async_copy(table_hbm.at[idx_v.at[chunk]], rows, sem)

    def drain(rows, sem):
        pltpu.make_async_copy(dummy_hbm, rows, sem).wait()

    def accumulate(rows, chunk):
        for bag in range(BAGS_PER_CHUNK):
            acc0 = [jnp.zeros((LANES,), jnp.float32)
                    for _ in range(KACC // LANES)]
            acc1 = [jnp.zeros((LANES,), jnp.float32)
                    for _ in range(KACC // LANES)]
            for j in range(0, HIST, 2):
                for k in range(KACC // LANES):
                    acc0[k] = acc0[k] + rows[bag * HIST + j,
                                             pl.ds(k * LANES, LANES)]
                    acc1[k] = acc1[k] + rows[bag * HIST + j + 1,
                                             pl.ds(k * LANES, LANES)]
            b_local = chunk * BAGS_PER_CHUNK + bag
            for k in range(KACC // LANES):
                out_v[b_local, pl.ds(k * LANES, LANES)] = acc0[k] + acc1[k]

    bufs = ((rows0, sem0), (rows1, sem1), (rows2, sem2), (rows3, sem3))
    nbuf = len(bufs)

    for k, (rows, sem) in enumerate(bufs):
        start(k, rows, sem)

    def body(c, _):
        for k, (rows, sem) in enumerate(bufs):
            drain(rows, sem)
            accumulate(rows, c + k)

            @pl.when(c + k + nbuf < NCH)
            def _():
                start(c + k + nbuf, rows, sem)
        return 0

    lax.fori_loop(0, NCH // nbuf, lambda i, carry: body(i * nbuf, carry), 0)

    pltpu.sync_copy(out_v, out_hbm.at[wid])


_sc_sum = functools.partial(
    pl.kernel,
    mesh=plsc.VectorSubcoreMesh(core_axis_name="c", subcore_axis_name="s",
                                num_cores=NC, num_subcores=NS),
    out_type=jax.ShapeDtypeStruct((NW, BPW, KACC), jnp.float32),
    scratch_types=[
        pltpu.VMEM((NCH, CIDX), jnp.int32),
        pltpu.VMEM((CIDX, KACC), jnp.float32),
        pltpu.VMEM((CIDX, KACC), jnp.float32),
        pltpu.VMEM((CIDX, KACC), jnp.float32),
        pltpu.VMEM((CIDX, KACC), jnp.float32),
        pltpu.VMEM((BPW, KACC), jnp.float32),
        pltpu.SemaphoreType.DMA,
        pltpu.SemaphoreType.DMA,
        pltpu.SemaphoreType.DMA,
        pltpu.SemaphoreType.DMA,
    ],
    compiler_params=pltpu.CompilerParams(use_tc_tiling_on_sc=False),
)(_sc_body)


def kernel(text, emb_weight, fc_weight, fc_bias):
    w32 = jnp.zeros((D, KACC), jnp.float32).at[:, :NUM_CLASS].set(
        fc_weight.T * (1.0 / HIST))
    b32 = jnp.zeros((1, KACC), jnp.float32).at[0, :NUM_CLASS].set(
        fc_bias * (1.0 / HIST))
    table = _proj(emb_weight.T, w32, b32).reshape(VOCAB_PAD, KACC)
    q_shift = (_VB // 4).bit_length() - 1
    v = text.astype(jnp.int32)
    w = (v & ~(_VB - 1)) + ((v & (_VB // 4 - 1)) << 2) + ((v >> q_shift) & 3)
    idx = w.reshape(NW, NCH, CIDX)
    dummy = jnp.zeros((CIDX, KACC), jnp.float32)
    sums = _sc_sum(idx, table, dummy).reshape(B, KACC)
    return sums[:, :NUM_CLASS]
